# baseline (speedup 1.0000x reference)
import functools

import jax
import jax.numpy as jnp
from jax import lax
from jax.experimental import pallas as pl
from jax.experimental.pallas import tpu as pltpu
from jax.experimental.pallas import tpu_sc as plsc

B, S, D = 2, 2048, 1024
E, FF = 8, 2048
BS = B * S
CAPACITY = 640.0
TB = 1024
NTB = BS // TB
BT = 256
NBLK = (BS * 2 + 16 + E * (BT - 1)) // BT + 1
PAD = NBLK * BT
DUMMY = PAD - 1
NW = 32
SB = 64
NSUB = 5
PPAD = NW * NSUB * SB
TPT = BS // NW
CB = 64


def _router_kernel(x_ref, wr_ref, br_ref, iw_ref, w_ref, a_ref):
    t = pl.program_id(0)

    @pl.when(t == 0)
    def _():
        a_ref[...] = jnp.zeros_like(a_ref)

    prev = a_ref[...]
    prevcnt = prev[2:3, :] + prev[3:4, :]

    x = x_ref[...]
    logits = lax.dot_general(
        x, wr_ref[...], (((1,), (1,)), ((), ())), preferred_element_type=jnp.float32
    ) + br_ref[...]
    mx = jnp.max(logits, axis=1, keepdims=True)
    ex = jnp.exp(logits - mx)
    probs = ex / jnp.sum(ex, axis=1, keepdims=True)
    iota_e = lax.broadcasted_iota(jnp.int32, (TB, E), 1)
    p1 = jnp.max(probs, axis=1, keepdims=True)
    i1 = jnp.argmax(probs, axis=1).reshape(TB, 1)
    masked = jnp.where(iota_e == i1, -jnp.inf, probs)
    p2 = jnp.max(masked, axis=1, keepdims=True)
    i2 = jnp.argmax(masked, axis=1).reshape(TB, 1)
    s = p1 + p2
    w1 = p1 / s
    w2 = p2 / s

    lane = lax.broadcasted_iota(jnp.int32, (TB, 128), 1)
    o1 = (lane == i1).astype(jnp.float32)
    o2 = (lane == i2).astype(jnp.float32)
    row_i = lax.broadcasted_iota(jnp.int32, (TB, TB), 0)
    col_i = lax.broadcasted_iota(jnp.int32, (TB, TB), 1)
    ltri = (row_i > col_i).astype(jnp.bfloat16)
    c1 = lax.dot_general(
        ltri, o1.astype(jnp.bfloat16), (((1,), (0,)), ((), ())),
        preferred_element_type=jnp.float32,
    )
    c2 = lax.dot_general(
        ltri, o2.astype(jnp.bfloat16), (((1,), (0,)), ((), ())),
        preferred_element_type=jnp.float32,
    )
    s1 = jnp.sum(o1, axis=0, keepdims=True)
    s2 = jnp.sum(o2, axis=0, keepdims=True)
    pos1 = jnp.sum((c1 + prevcnt) * o1, axis=1, keepdims=True)
    pos2 = jnp.sum((c2 + prevcnt + s1) * o2, axis=1, keepdims=True)

    i1f = i1.astype(jnp.int32)
    i2f = i2.astype(jnp.int32)
    iw = (
        jnp.where(lane == 0, i1f, 0)
        + jnp.where(lane == 1, i2f, 0)
        + jnp.where(lane == 2, pos1.astype(jnp.int32), 0)
        + jnp.where(lane == 3, pos2.astype(jnp.int32), 0)
    )
    iw_ref[...] = iw
    w_ref[...] = jnp.where(lane == 0, w1, 0.0) + jnp.where(lane == 1, w2, 0.0)

    a1 = jnp.sum(w1 * o1, axis=0, keepdims=True)
    a2 = jnp.sum(w2 * o2, axis=0, keepdims=True)
    srow = lax.broadcasted_iota(jnp.int32, (8, 128), 0)
    delta = (
        jnp.where(srow == 0, a1, 0.0)
        + jnp.where(srow == 1, a2, 0.0)
        + jnp.where(srow == 2, s1, 0.0)
        + jnp.where(srow == 3, s2, 0.0)
    )
    a_ref[...] += delta


def _ffn_kernel(em_ref, rows_ref, val_ref, w1_ref, b1_ref, w2_ref, b2_ref, o_ref):
    del em_ref
    v = val_ref[...]
    xs = (rows_ref[...].astype(jnp.float32) * v).astype(jnp.bfloat16)
    h = lax.dot_general(
        xs, w1_ref[0], (((1,), (1,)), ((), ())), preferred_element_type=jnp.float32
    ) + b1_ref[0]
    h = 0.5 * h * (1.0 + lax.erf(h * 0.7071067811865476))
    out = lax.dot_general(
        h.astype(jnp.bfloat16), w2_ref[0], (((1,), (1,)), ((), ())),
        preferred_element_type=jnp.float32,
    ) + b2_ref[0]
    o_ref[...] = (v * out).astype(jnp.bfloat16)


RPT = PAD // NW
GSB = 64
NGS = RPT // GSB + 1


def _sc_dispatch(x3, src, dst, val):
    info = plsc.get_sparse_core_info()
    nc = info.num_cores
    mesh = plsc.VectorSubcoreMesh(core_axis_name="c", subcore_axis_name="s")

    @functools.partial(
        pl.kernel,
        out_type=[
            jax.ShapeDtypeStruct((PAD, 4, 128), jnp.int32),
            jax.ShapeDtypeStruct((PAD,), jnp.float32),
        ],
        mesh=mesh,
        scratch_types=[
            pltpu.VMEM((PPAD,), jnp.int32),
            pltpu.VMEM((PPAD,), jnp.int32),
            pltpu.VMEM((PPAD,), jnp.float32),
            pltpu.VMEM((PAD,), jnp.int32),
            pltpu.VMEM((PAD,), jnp.float32),
            pltpu.VMEM((GSB, 4, 128), jnp.int32),
            pltpu.VMEM((GSB, 4, 128), jnp.int32),
            pltpu.SemaphoreType.DMA,
            pltpu.SemaphoreType.DMA,
        ],
        compiler_params=pltpu.CompilerParams(needs_layout_passes=False),
    )
    def k(x_hbm, src_hbm, dst_hbm, val_hbm, rows_out, vbuf_out,
          src_v, dst_v, val_v, spos_v, vpos_v, rows_a, rows_b, sem_a, sem_b):
        wid = lax.axis_index("s") * nc + lax.axis_index("c")
        pltpu.sync_copy(src_hbm, src_v)
        pltpu.sync_copy(dst_hbm, dst_v)
        pltpu.sync_copy(val_hbm, val_v)
        base = wid * RPT

        zstart = jnp.minimum((base // 16) * 16, PAD - 22 * 16)

        def zero_body(i, _):
            spos_v[pl.ds(zstart + i * 16, 16)] = jnp.zeros((16,), jnp.int32)
            vpos_v[pl.ds(zstart + i * 16, 16)] = jnp.zeros((16,), jnp.float32)
            return 0

        lax.fori_loop(0, 22, zero_body, 0)

        def inv_body(i, _):
            for u in range(4):
                o = pl.ds((i * 4 + u) * 16, 16)
                idx = dst_v[o]
                plsc.store_scatter(spos_v, [idx], src_v[o])
                plsc.store_scatter(vpos_v, [idx], val_v[o])
            return 0

        lax.fori_loop(0, PPAD // 64, inv_body, 0)

        pltpu.sync_copy(vpos_v.at[pl.ds(base, RPT)], vbuf_out.at[pl.ds(base, RPT)])
        bufs = [rows_a, rows_b]
        sems = [sem_a, sem_b]
        sizes = [GSB] * (NGS - 1) + [RPT - (NGS - 1) * GSB]

        def start(j):
            return pltpu.async_copy(
                x_hbm.at[spos_v.at[pl.ds(base + j * GSB, sizes[j])]],
                bufs[j % 2].at[pl.ds(0, sizes[j])], sems[j % 2],
            )

        cp = start(0)
        for j in range(NGS):
            cp.wait()
            if j + 1 < NGS:
                cp = start(j + 1)
            pltpu.sync_copy(
                bufs[j % 2].at[pl.ds(0, sizes[j])],
                rows_out.at[pl.ds(base + j * GSB, sizes[j])],
            )

    return k(x3, src, dst, val)


def _sc_combine(raw, g01, gx):
    info = plsc.get_sparse_core_info()
    nc = info.num_cores
    mesh = plsc.VectorSubcoreMesh(core_axis_name="c", subcore_axis_name="s")

    @functools.partial(
        pl.kernel,
        out_type=[
            jax.ShapeDtypeStruct((BS, 4, 128), jnp.int32),
            jax.ShapeDtypeStruct((BS, 4, 128), jnp.int32),
            jax.ShapeDtypeStruct((16, 4, 128), jnp.int32),
        ],
        mesh=mesh,
        scratch_types=[
            pltpu.VMEM((2, TPT), jnp.int32),
            pltpu.VMEM((2, 8), jnp.int32),
            pltpu.VMEM((CB, 4, 128), jnp.int32),
            pltpu.SemaphoreType.DMA,
        ],
    )
    def k(raw_hbm, g01_hbm, gx_hbm, out0_hbm, out1_hbm, ex_hbm,
          g_v, gx_v, buf_v, sem):
        wid = lax.axis_index("s") * nc + lax.axis_index("c")
        pltpu.sync_copy(g01_hbm.at[wid], g_v)
        for r in range(TPT // CB):
            base = wid * TPT + r * CB
            pltpu.async_copy(
                raw_hbm.at[g_v.at[0, pl.ds(r * CB, CB)]], buf_v, sem
            ).wait()
            pltpu.sync_copy(buf_v, out0_hbm.at[pl.ds(base, CB)])
            pltpu.async_copy(
                raw_hbm.at[g_v.at[1, pl.ds(r * CB, CB)]], buf_v, sem
            ).wait()
            pltpu.sync_copy(buf_v, out1_hbm.at[pl.ds(base, CB)])

        @pl.when(wid == 0)
        def _():
            pltpu.sync_copy(gx_hbm, gx_v)
            pltpu.async_copy(
                raw_hbm.at[gx_v.at[0]], buf_v.at[pl.ds(0, 8)], sem
            ).wait()
            pltpu.sync_copy(buf_v.at[pl.ds(0, 8)], ex_hbm.at[pl.ds(0, 8)])
            pltpu.async_copy(
                raw_hbm.at[gx_v.at[1]], buf_v.at[pl.ds(0, 8)], sem
            ).wait()
            pltpu.sync_copy(buf_v.at[pl.ds(0, 8)], ex_hbm.at[pl.ds(8, 8)])

    return k(raw, g01, gx)


def _fin_kernel(g0_ref, g1_ref, ex_ref, o_ref):
    blk = pl.program_id(0)
    o_ref[...] = g0_ref[...].astype(jnp.float32) + g1_ref[...].astype(jnp.float32)

    @pl.when(blk == 0)
    def _():
        o_ref[0:8, :] = o_ref[0:8, :] + (
            ex_ref[0:8, :].astype(jnp.float32) + ex_ref[8:16, :].astype(jnp.float32)
        )


@jax.jit
def kernel(inputs, Wr, br, W1, b1, W2, b2):
    b, s, d = inputs.shape
    xf = inputs.reshape(BS, D)

    iw, w, a = pl.pallas_call(
        _router_kernel,
        grid=(NTB,),
        in_specs=[
            pl.BlockSpec((TB, D), lambda t: (t, 0)),
            pl.BlockSpec((E, D), lambda t: (0, 0)),
            pl.BlockSpec((1, E), lambda t: (0, 0)),
        ],
        out_specs=[
            pl.BlockSpec((TB, 128), lambda t: (t, 0)),
            pl.BlockSpec((TB, 128), lambda t: (t, 0)),
            pl.BlockSpec((8, 128), lambda t: (0, 0)),
        ],
        out_shape=[
            jax.ShapeDtypeStruct((BS, 128), jnp.int32),
            jax.ShapeDtypeStruct((BS, 128), jnp.float32),
            jax.ShapeDtypeStruct((8, 128), jnp.float32),
        ],
    )(xf, Wr, br.reshape(1, E))

    i1, i2 = iw[:, 0], iw[:, 1]
    pos1, pos2 = iw[:, 2], iw[:, 3]
    w1v, w2v = w[:, 0], w[:, 1]
    amass = a[0:2, 0:E]
    n_slots = (a[2, 0:E] + a[3, 0:E]).astype(jnp.int32)
    n_tot = n_slots + jnp.where(jnp.arange(E) < 2, 8, 0)
    padded = ((n_tot + BT - 1) // BT) * BT
    off = jnp.concatenate([jnp.zeros((1,), jnp.int32), jnp.cumsum(padded)[:-1]])
    cum_blk = jnp.cumsum(padded // BT)
    blk_expert = jnp.minimum(
        jnp.sum(
            (jnp.arange(NBLK, dtype=jnp.int32)[:, None] >= cum_blk[None, :]).astype(
                jnp.int32
            ),
            axis=1,
        ),
        E - 1,
    ).astype(jnp.int32)

    r8 = jnp.arange(8)
    i1_8, i2_8 = i1[:8], i2[:8]
    c0 = jnp.where(i1_8 < 2, amass[jnp.clip(i1_8, 0, 1), r8], 0.0)
    c1 = jnp.where(i2_8 < 2, amass[jnp.clip(i2_8, 0, 1), r8], 0.0)
    v0 = jnp.minimum(w1v.at[0:8].add(c0), CAPACITY)
    v1 = jnp.minimum(w2v.at[0:8].add(c1), CAPACITY)
    in_top = (i1_8[:, None] == jnp.arange(2)[None, :]) | (
        i2_8[:, None] == jnp.arange(2)[None, :]
    )
    vx = jnp.where(in_top, 0.0, jnp.minimum(amass.T, CAPACITY))
    dx = off[None, :2] + n_slots[None, :2] + r8[:, None]

    d0 = off[i1] + pos1
    d1 = off[i2] + pos2
    npad = PPAD - (2 * BS + 16)
    toks = jnp.arange(BS, dtype=jnp.int32)
    src = jnp.concatenate(
        [toks, toks, jnp.broadcast_to(r8[:, None], (8, 2)).reshape(-1),
         jnp.zeros((npad,), jnp.int32)]
    )
    dst = jnp.concatenate(
        [d0, d1, dx.reshape(-1), jnp.full((npad,), DUMMY, jnp.int32)]
    )
    val = jnp.concatenate(
        [v0, v1, vx.reshape(-1), jnp.zeros((npad,), jnp.float32)]
    )

    x3 = lax.bitcast_convert_type(
        xf.astype(jnp.bfloat16).reshape(BS, 4, 128, 2), jnp.int32
    )
    rows3, vbuf = _sc_dispatch(x3, src, dst, val)
    rows = lax.bitcast_convert_type(rows3, jnp.bfloat16).reshape(PAD, D)

    w1b = W1.astype(jnp.bfloat16)
    w2b = W2.astype(jnp.bfloat16)
    raw = pl.pallas_call(
        _ffn_kernel,
        grid_spec=pltpu.PrefetchScalarGridSpec(
            num_scalar_prefetch=1,
            grid=(NBLK,),
            in_specs=[
                pl.BlockSpec((BT, D), lambda i, em: (i, 0)),
                pl.BlockSpec((BT, 1), lambda i, em: (i, 0)),
                pl.BlockSpec((1, FF, D), lambda i, em: (em[i], 0, 0)),
                pl.BlockSpec((1, 1, FF), lambda i, em: (em[i], 0, 0)),
                pl.BlockSpec((1, D, FF), lambda i, em: (em[i], 0, 0)),
                pl.BlockSpec((1, 1, D), lambda i, em: (em[i], 0, 0)),
            ],
            out_specs=pl.BlockSpec((BT, D), lambda i, em: (i, 0)),
        ),
        out_shape=jax.ShapeDtypeStruct((PAD, D), jnp.bfloat16),
    )(
        blk_expert, rows, vbuf.reshape(PAD, 1), w1b, b1.reshape(E, 1, FF),
        w2b, b2.reshape(E, 1, D),
    )

    g01 = jnp.stack([d0, d1]).reshape(2, NW, TPT).transpose(1, 0, 2)
    gx = dx.T.astype(jnp.int32)

    raw_i = lax.bitcast_convert_type(raw.reshape(PAD, 4, 128, 2), jnp.int32)
    g0rows, g1rows, exrows = _sc_combine(raw_i, g01, gx)
    FB = 512
    out = pl.pallas_call(
        _fin_kernel,
        grid=(BS // FB,),
        in_specs=[
            pl.BlockSpec((FB, D), lambda i: (i, 0)),
            pl.BlockSpec((FB, D), lambda i: (i, 0)),
            pl.BlockSpec((16, D), lambda i: (0, 0)),
        ],
        out_specs=pl.BlockSpec((FB, D), lambda i: (i, 0)),
        out_shape=jax.ShapeDtypeStruct((BS, D), jnp.float32),
    )(
        lax.bitcast_convert_type(g0rows, jnp.bfloat16).reshape(BS, D),
        lax.bitcast_convert_type(g1rows, jnp.bfloat16).reshape(BS, D),
        lax.bitcast_convert_type(exrows, jnp.bfloat16).reshape(16, D),
    )
    return out.reshape(b, s, d)


# speedup vs baseline: 2.5242x; 2.5242x over previous
import functools

import jax
import jax.numpy as jnp
from jax import lax
from jax.experimental import pallas as pl
from jax.experimental.pallas import tpu as pltpu
from jax.experimental.pallas import tpu_sc as plsc

B, S, D = 2, 2048, 1024
E, FF = 8, 2048
BS = B * S
CAPACITY = 640.0
TB = 1024
NTB = BS // TB
BT = 256
NBLK = (BS * 2 + 16 + E * (BT - 1)) // BT + 1
PAD = NBLK * BT
DUMMY = PAD - 1
NW = 32
SB = 64
NSUB = 5
PPAD = NW * NSUB * SB
TPT = BS // NW
CB = 64


def _router_kernel(x_ref, wr_ref, br_ref, iw_ref, w_ref, a_ref):
    t = pl.program_id(0)

    @pl.when(t == 0)
    def _():
        a_ref[...] = jnp.zeros_like(a_ref)

    prev = a_ref[...]
    prevcnt = prev[2:3, :] + prev[3:4, :]

    x = x_ref[...]
    logits = lax.dot_general(
        x, wr_ref[...], (((1,), (1,)), ((), ())), preferred_element_type=jnp.float32
    ) + br_ref[...]
    mx = jnp.max(logits, axis=1, keepdims=True)
    ex = jnp.exp(logits - mx)
    probs = ex / jnp.sum(ex, axis=1, keepdims=True)
    iota_e = lax.broadcasted_iota(jnp.int32, (TB, E), 1)
    p1 = jnp.max(probs, axis=1, keepdims=True)
    i1 = jnp.argmax(probs, axis=1).reshape(TB, 1)
    masked = jnp.where(iota_e == i1, -jnp.inf, probs)
    p2 = jnp.max(masked, axis=1, keepdims=True)
    i2 = jnp.argmax(masked, axis=1).reshape(TB, 1)
    s = p1 + p2
    w1 = p1 / s
    w2 = p2 / s

    lane = lax.broadcasted_iota(jnp.int32, (TB, 128), 1)
    o1 = (lane == i1).astype(jnp.float32)
    o2 = (lane == i2).astype(jnp.float32)
    row_i = lax.broadcasted_iota(jnp.int32, (TB, TB), 0)
    col_i = lax.broadcasted_iota(jnp.int32, (TB, TB), 1)
    ltri = (row_i > col_i).astype(jnp.bfloat16)
    c1 = lax.dot_general(
        ltri, o1.astype(jnp.bfloat16), (((1,), (0,)), ((), ())),
        preferred_element_type=jnp.float32,
    )
    c2 = lax.dot_general(
        ltri, o2.astype(jnp.bfloat16), (((1,), (0,)), ((), ())),
        preferred_element_type=jnp.float32,
    )
    s1 = jnp.sum(o1, axis=0, keepdims=True)
    s2 = jnp.sum(o2, axis=0, keepdims=True)
    pos1 = jnp.sum((c1 + prevcnt) * o1, axis=1, keepdims=True)
    pos2 = jnp.sum((c2 + prevcnt + s1) * o2, axis=1, keepdims=True)

    i1f = i1.astype(jnp.int32)
    i2f = i2.astype(jnp.int32)
    iw = (
        jnp.where(lane == 0, i1f, 0)
        + jnp.where(lane == 1, i2f, 0)
        + jnp.where(lane == 2, pos1.astype(jnp.int32), 0)
        + jnp.where(lane == 3, pos2.astype(jnp.int32), 0)
    )
    iw_ref[...] = iw
    w_ref[...] = jnp.where(lane == 0, w1, 0.0) + jnp.where(lane == 1, w2, 0.0)

    a1 = jnp.sum(w1 * o1, axis=0, keepdims=True)
    a2 = jnp.sum(w2 * o2, axis=0, keepdims=True)
    srow = lax.broadcasted_iota(jnp.int32, (8, 128), 0)
    delta = (
        jnp.where(srow == 0, a1, 0.0)
        + jnp.where(srow == 1, a2, 0.0)
        + jnp.where(srow == 2, s1, 0.0)
        + jnp.where(srow == 3, s2, 0.0)
    )
    a_ref[...] += delta


def _ffn_kernel(em_ref, rows_ref, val_ref, w1_ref, b1_ref, w2_ref, b2_ref, o_ref):
    del em_ref
    v = val_ref[...]
    xs = (rows_ref[...] * v).astype(jnp.bfloat16)
    h = lax.dot_general(
        xs, w1_ref[0], (((1,), (1,)), ((), ())), preferred_element_type=jnp.float32
    ) + b1_ref[0]
    h = 0.5 * h * (1.0 + lax.erf(h * 0.7071067811865476))
    out = lax.dot_general(
        h.astype(jnp.bfloat16), w2_ref[0], (((1,), (1,)), ((), ())),
        preferred_element_type=jnp.float32,
    ) + b2_ref[0]
    o_ref[...] = v * out


RPT = PAD // NW
GSB = 32
NGS = RPT // GSB + 1


def _sc_dispatch(x3, src, dst, val):
    info = plsc.get_sparse_core_info()
    nc = info.num_cores
    mesh = plsc.VectorSubcoreMesh(core_axis_name="c", subcore_axis_name="s")

    @functools.partial(
        pl.kernel,
        out_type=[
            jax.ShapeDtypeStruct((PAD, D), jnp.float32),
            jax.ShapeDtypeStruct((PAD,), jnp.float32),
        ],
        mesh=mesh,
        scratch_types=[
            pltpu.VMEM((PPAD,), jnp.int32),
            pltpu.VMEM((PPAD,), jnp.int32),
            pltpu.VMEM((PPAD,), jnp.float32),
            pltpu.VMEM((PAD,), jnp.int32),
            pltpu.VMEM((PAD,), jnp.float32),
            pltpu.VMEM((GSB, D), jnp.float32),
            pltpu.VMEM((GSB, D), jnp.float32),
            pltpu.SemaphoreType.DMA,
            pltpu.SemaphoreType.DMA,
        ],
        compiler_params=pltpu.CompilerParams(needs_layout_passes=False),
    )
    def k(x_hbm, src_hbm, dst_hbm, val_hbm, rows_out, vbuf_out,
          src_v, dst_v, val_v, spos_v, vpos_v, rows_a, rows_b, sem_a, sem_b):
        wid = lax.axis_index("s") * nc + lax.axis_index("c")
        pltpu.sync_copy(src_hbm, src_v)
        pltpu.sync_copy(dst_hbm, dst_v)
        pltpu.sync_copy(val_hbm, val_v)
        base = wid * RPT

        zstart = jnp.minimum((base // 16) * 16, PAD - 22 * 16)

        def zero_body(i, _):
            spos_v[pl.ds(zstart + i * 16, 16)] = jnp.zeros((16,), jnp.int32)
            vpos_v[pl.ds(zstart + i * 16, 16)] = jnp.zeros((16,), jnp.float32)
            return 0

        lax.fori_loop(0, 22, zero_body, 0)

        def inv_body(i, _):
            for u in range(4):
                o = pl.ds((i * 4 + u) * 16, 16)
                idx = dst_v[o]
                plsc.store_scatter(spos_v, [idx], src_v[o])
                plsc.store_scatter(vpos_v, [idx], val_v[o])
            return 0

        lax.fori_loop(0, PPAD // 64, inv_body, 0)

        pltpu.sync_copy(vpos_v.at[pl.ds(base, RPT)], vbuf_out.at[pl.ds(base, RPT)])
        bufs = [rows_a, rows_b]
        sems = [sem_a, sem_b]
        sizes = [GSB] * (NGS - 1) + [RPT - (NGS - 1) * GSB]

        def start(j):
            return pltpu.async_copy(
                x_hbm.at[spos_v.at[pl.ds(base + j * GSB, sizes[j])]],
                bufs[j % 2].at[pl.ds(0, sizes[j])], sems[j % 2],
            )

        cp = start(0)
        for j in range(NGS):
            cp.wait()
            if j + 1 < NGS:
                cp = start(j + 1)
            pltpu.sync_copy(
                bufs[j % 2].at[pl.ds(0, sizes[j])],
                rows_out.at[pl.ds(base + j * GSB, sizes[j])],
            )

    return k(x3, src, dst, val)


def _sc_combine(raw, g01, gx):
    info = plsc.get_sparse_core_info()
    nc = info.num_cores
    mesh = plsc.VectorSubcoreMesh(core_axis_name="c", subcore_axis_name="s")

    @functools.partial(
        pl.kernel,
        out_type=[
            jax.ShapeDtypeStruct((BS, D), jnp.float32),
            jax.ShapeDtypeStruct((BS, D), jnp.float32),
            jax.ShapeDtypeStruct((16, D), jnp.float32),
        ],
        mesh=mesh,
        scratch_types=[
            pltpu.VMEM((2, TPT), jnp.int32),
            pltpu.VMEM((2, 8), jnp.int32),
            pltpu.VMEM((CB, D), jnp.float32),
            pltpu.SemaphoreType.DMA,
        ],
    )
    def k(raw_hbm, g01_hbm, gx_hbm, out0_hbm, out1_hbm, ex_hbm,
          g_v, gx_v, buf_v, sem):
        wid = lax.axis_index("s") * nc + lax.axis_index("c")
        pltpu.sync_copy(g01_hbm.at[wid], g_v)
        for r in range(TPT // CB):
            base = wid * TPT + r * CB
            pltpu.async_copy(
                raw_hbm.at[g_v.at[0, pl.ds(r * CB, CB)]], buf_v, sem
            ).wait()
            pltpu.sync_copy(buf_v, out0_hbm.at[pl.ds(base, CB)])
            pltpu.async_copy(
                raw_hbm.at[g_v.at[1, pl.ds(r * CB, CB)]], buf_v, sem
            ).wait()
            pltpu.sync_copy(buf_v, out1_hbm.at[pl.ds(base, CB)])

        @pl.when(wid == 0)
        def _():
            pltpu.sync_copy(gx_hbm, gx_v)
            pltpu.async_copy(
                raw_hbm.at[gx_v.at[0]], buf_v.at[pl.ds(0, 8)], sem
            ).wait()
            pltpu.sync_copy(buf_v.at[pl.ds(0, 8)], ex_hbm.at[pl.ds(0, 8)])
            pltpu.async_copy(
                raw_hbm.at[gx_v.at[1]], buf_v.at[pl.ds(0, 8)], sem
            ).wait()
            pltpu.sync_copy(buf_v.at[pl.ds(0, 8)], ex_hbm.at[pl.ds(8, 8)])

    return k(raw, g01, gx)


def _fin_kernel(g0_ref, g1_ref, ex_ref, o_ref):
    blk = pl.program_id(0)
    o_ref[...] = g0_ref[...] + g1_ref[...]

    @pl.when(blk == 0)
    def _():
        o_ref[0:8, :] = o_ref[0:8, :] + (ex_ref[0:8, :] + ex_ref[8:16, :])


@jax.jit
def kernel(inputs, Wr, br, W1, b1, W2, b2):
    b, s, d = inputs.shape
    xf = inputs.reshape(BS, D)

    iw, w, a = pl.pallas_call(
        _router_kernel,
        grid=(NTB,),
        in_specs=[
            pl.BlockSpec((TB, D), lambda t: (t, 0)),
            pl.BlockSpec((E, D), lambda t: (0, 0)),
            pl.BlockSpec((1, E), lambda t: (0, 0)),
        ],
        out_specs=[
            pl.BlockSpec((TB, 128), lambda t: (t, 0)),
            pl.BlockSpec((TB, 128), lambda t: (t, 0)),
            pl.BlockSpec((8, 128), lambda t: (0, 0)),
        ],
        out_shape=[
            jax.ShapeDtypeStruct((BS, 128), jnp.int32),
            jax.ShapeDtypeStruct((BS, 128), jnp.float32),
            jax.ShapeDtypeStruct((8, 128), jnp.float32),
        ],
    )(xf, Wr, br.reshape(1, E))

    i1, i2 = iw[:, 0], iw[:, 1]
    pos1, pos2 = iw[:, 2], iw[:, 3]
    w1v, w2v = w[:, 0], w[:, 1]
    amass = a[0:2, 0:E]
    n_slots = (a[2, 0:E] + a[3, 0:E]).astype(jnp.int32)
    n_tot = n_slots + jnp.where(jnp.arange(E) < 2, 8, 0)
    padded = ((n_tot + BT - 1) // BT) * BT
    off = jnp.concatenate([jnp.zeros((1,), jnp.int32), jnp.cumsum(padded)[:-1]])
    cum_blk = jnp.cumsum(padded // BT)
    blk_expert = jnp.minimum(
        jnp.sum(
            (jnp.arange(NBLK, dtype=jnp.int32)[:, None] >= cum_blk[None, :]).astype(
                jnp.int32
            ),
            axis=1,
        ),
        E - 1,
    ).astype(jnp.int32)

    r8 = jnp.arange(8)
    i1_8, i2_8 = i1[:8], i2[:8]
    c0 = jnp.where(i1_8 < 2, amass[jnp.clip(i1_8, 0, 1), r8], 0.0)
    c1 = jnp.where(i2_8 < 2, amass[jnp.clip(i2_8, 0, 1), r8], 0.0)
    v0 = jnp.minimum(w1v.at[0:8].add(c0), CAPACITY)
    v1 = jnp.minimum(w2v.at[0:8].add(c1), CAPACITY)
    in_top = (i1_8[:, None] == jnp.arange(2)[None, :]) | (
        i2_8[:, None] == jnp.arange(2)[None, :]
    )
    vx = jnp.where(in_top, 0.0, jnp.minimum(amass.T, CAPACITY))
    dx = off[None, :2] + n_slots[None, :2] + r8[:, None]

    d0 = off[i1] + pos1
    d1 = off[i2] + pos2
    npad = PPAD - (2 * BS + 16)
    toks = jnp.arange(BS, dtype=jnp.int32)
    src = jnp.concatenate(
        [toks, toks, jnp.broadcast_to(r8[:, None], (8, 2)).reshape(-1),
         jnp.zeros((npad,), jnp.int32)]
    )
    dst = jnp.concatenate(
        [d0, d1, dx.reshape(-1), jnp.full((npad,), DUMMY, jnp.int32)]
    )
    val = jnp.concatenate(
        [v0, v1, vx.reshape(-1), jnp.zeros((npad,), jnp.float32)]
    )

    rows, vbuf = _sc_dispatch(xf, src, dst, val)

    w1b = W1.astype(jnp.bfloat16)
    w2b = W2.astype(jnp.bfloat16)
    raw = pl.pallas_call(
        _ffn_kernel,
        grid_spec=pltpu.PrefetchScalarGridSpec(
            num_scalar_prefetch=1,
            grid=(NBLK,),
            in_specs=[
                pl.BlockSpec((BT, D), lambda i, em: (i, 0)),
                pl.BlockSpec((BT, 1), lambda i, em: (i, 0)),
                pl.BlockSpec((1, FF, D), lambda i, em: (em[i], 0, 0)),
                pl.BlockSpec((1, 1, FF), lambda i, em: (em[i], 0, 0)),
                pl.BlockSpec((1, D, FF), lambda i, em: (em[i], 0, 0)),
                pl.BlockSpec((1, 1, D), lambda i, em: (em[i], 0, 0)),
            ],
            out_specs=pl.BlockSpec((BT, D), lambda i, em: (i, 0)),
        ),
        out_shape=jax.ShapeDtypeStruct((PAD, D), jnp.float32),
    )(
        blk_expert, rows, vbuf.reshape(PAD, 1), w1b, b1.reshape(E, 1, FF),
        w2b, b2.reshape(E, 1, D),
    )

    g01 = jnp.stack([d0, d1]).reshape(2, NW, TPT).transpose(1, 0, 2)
    gx = dx.T.astype(jnp.int32)

    g0rows, g1rows, exrows = _sc_combine(raw, g01, gx)
    FB = 512
    out = pl.pallas_call(
        _fin_kernel,
        grid=(BS // FB,),
        in_specs=[
            pl.BlockSpec((FB, D), lambda i: (i, 0)),
            pl.BlockSpec((FB, D), lambda i: (i, 0)),
            pl.BlockSpec((16, D), lambda i: (0, 0)),
        ],
        out_specs=pl.BlockSpec((FB, D), lambda i: (i, 0)),
        out_shape=jax.ShapeDtypeStruct((BS, D), jnp.float32),
    )(g0rows, g1rows, exrows)
    return out.reshape(b, s, d)


# speedup vs baseline: 3.4675x; 1.3737x over previous
import functools

import jax
import jax.numpy as jnp
from jax import lax
from jax.experimental import pallas as pl
from jax.experimental.pallas import tpu as pltpu
from jax.experimental.pallas import tpu_sc as plsc

B, S, D = 2, 2048, 1024
E, FF = 8, 2048
BS = B * S
CAPACITY = 640.0
TB = 1024
NTB = BS // TB
BT = 256
NBLK = (BS * 2 + 16 + E * (BT - 1)) // BT + 1
PAD = NBLK * BT
DUMMY = PAD - 1
NW = 32
SB = 64
NSUB = 5
PPAD = NW * NSUB * SB
TPT = BS // NW
CB = 64


def _router_kernel(x_ref, wr_ref, br_ref, iw_ref, w_ref, a_ref):
    t = pl.program_id(0)

    @pl.when(t == 0)
    def _():
        a_ref[...] = jnp.zeros_like(a_ref)

    prev = a_ref[...]
    prevcnt = prev[2:3, :] + prev[3:4, :]

    x = x_ref[...]
    logits = lax.dot_general(
        x, wr_ref[...], (((1,), (1,)), ((), ())), preferred_element_type=jnp.float32
    ) + br_ref[...]
    mx = jnp.max(logits, axis=1, keepdims=True)
    ex = jnp.exp(logits - mx)
    probs = ex / jnp.sum(ex, axis=1, keepdims=True)
    iota_e = lax.broadcasted_iota(jnp.int32, (TB, E), 1)
    p1 = jnp.max(probs, axis=1, keepdims=True)
    i1 = jnp.argmax(probs, axis=1).reshape(TB, 1)
    masked = jnp.where(iota_e == i1, -jnp.inf, probs)
    p2 = jnp.max(masked, axis=1, keepdims=True)
    i2 = jnp.argmax(masked, axis=1).reshape(TB, 1)
    s = p1 + p2
    w1 = p1 / s
    w2 = p2 / s

    lane = lax.broadcasted_iota(jnp.int32, (TB, 128), 1)
    o1 = (lane == i1).astype(jnp.float32)
    o2 = (lane == i2).astype(jnp.float32)
    row_i = lax.broadcasted_iota(jnp.int32, (TB, TB), 0)
    col_i = lax.broadcasted_iota(jnp.int32, (TB, TB), 1)
    ltri = (row_i > col_i).astype(jnp.bfloat16)
    c1 = lax.dot_general(
        ltri, o1.astype(jnp.bfloat16), (((1,), (0,)), ((), ())),
        preferred_element_type=jnp.float32,
    )
    c2 = lax.dot_general(
        ltri, o2.astype(jnp.bfloat16), (((1,), (0,)), ((), ())),
        preferred_element_type=jnp.float32,
    )
    s1 = jnp.sum(o1, axis=0, keepdims=True)
    s2 = jnp.sum(o2, axis=0, keepdims=True)
    pos1 = jnp.sum((c1 + prevcnt) * o1, axis=1, keepdims=True)
    pos2 = jnp.sum((c2 + prevcnt + s1) * o2, axis=1, keepdims=True)

    i1f = i1.astype(jnp.int32)
    i2f = i2.astype(jnp.int32)
    iw = (
        jnp.where(lane == 0, i1f, 0)
        + jnp.where(lane == 1, i2f, 0)
        + jnp.where(lane == 2, pos1.astype(jnp.int32), 0)
        + jnp.where(lane == 3, pos2.astype(jnp.int32), 0)
    )
    iw_ref[...] = iw
    w_ref[...] = jnp.where(lane == 0, w1, 0.0) + jnp.where(lane == 1, w2, 0.0)

    a1 = jnp.sum(w1 * o1, axis=0, keepdims=True)
    a2 = jnp.sum(w2 * o2, axis=0, keepdims=True)
    srow = lax.broadcasted_iota(jnp.int32, (8, 128), 0)
    delta = (
        jnp.where(srow == 0, a1, 0.0)
        + jnp.where(srow == 1, a2, 0.0)
        + jnp.where(srow == 2, s1, 0.0)
        + jnp.where(srow == 3, s2, 0.0)
    )
    a_ref[...] += delta


def _ffn_kernel(em_ref, rows_ref, val_ref, w1_ref, b1_ref, w2_ref, b2_ref, o_ref):
    del em_ref
    v = val_ref[...]
    xs = (rows_ref[...] * v).astype(jnp.bfloat16)
    h = lax.dot_general(
        xs, w1_ref[0], (((1,), (1,)), ((), ())), preferred_element_type=jnp.float32
    ) + b1_ref[0]
    h = 0.5 * h * (1.0 + lax.erf(h * 0.7071067811865476))
    out = lax.dot_general(
        h.astype(jnp.bfloat16), w2_ref[0], (((1,), (1,)), ((), ())),
        preferred_element_type=jnp.float32,
    ) + b2_ref[0]
    o_ref[...] = v * out


RPT = PAD // NW
GSB = 32
NGS = RPT // GSB + 1


def _sc_dispatch(x3, src, dst, val):
    info = plsc.get_sparse_core_info()
    nc = info.num_cores
    mesh = plsc.VectorSubcoreMesh(core_axis_name="c", subcore_axis_name="s")

    @functools.partial(
        pl.kernel,
        out_type=[
            jax.ShapeDtypeStruct((PAD, D), jnp.float32),
            jax.ShapeDtypeStruct((PAD,), jnp.float32),
        ],
        mesh=mesh,
        scratch_types=[
            pltpu.VMEM((PPAD,), jnp.int32),
            pltpu.VMEM((PPAD,), jnp.int32),
            pltpu.VMEM((PPAD,), jnp.float32),
            pltpu.VMEM((PAD,), jnp.int32),
            pltpu.VMEM((PAD,), jnp.float32),
            pltpu.VMEM((GSB, D), jnp.float32),
            pltpu.VMEM((GSB, D), jnp.float32),
            pltpu.SemaphoreType.DMA,
            pltpu.SemaphoreType.DMA,
        ],
        compiler_params=pltpu.CompilerParams(needs_layout_passes=False),
    )
    def k(x_hbm, src_hbm, dst_hbm, val_hbm, rows_out, vbuf_out,
          src_v, dst_v, val_v, spos_v, vpos_v, rows_a, rows_b, sem_a, sem_b):
        wid = lax.axis_index("s") * nc + lax.axis_index("c")
        pltpu.sync_copy(src_hbm, src_v)
        pltpu.sync_copy(dst_hbm, dst_v)
        pltpu.sync_copy(val_hbm, val_v)
        base = wid * RPT

        zstart = jnp.minimum((base // 16) * 16, PAD - 22 * 16)

        def zero_body(i, _):
            off = zstart + i * 16
            spos_v[pl.ds(off, 16)] = (lax.iota(jnp.int32, 16) + off) & (BS - 1)
            vpos_v[pl.ds(off, 16)] = jnp.zeros((16,), jnp.float32)
            return 0

        lax.fori_loop(0, 22, zero_body, 0)

        def inv_body(i, _):
            for u in range(4):
                o = pl.ds((i * 4 + u) * 16, 16)
                idx = dst_v[o]
                plsc.store_scatter(spos_v, [idx], src_v[o])
                plsc.store_scatter(vpos_v, [idx], val_v[o])
            return 0

        lax.fori_loop(0, PPAD // 64, inv_body, 0)

        pltpu.sync_copy(vpos_v.at[pl.ds(base, RPT)], vbuf_out.at[pl.ds(base, RPT)])
        bufs = [rows_a, rows_b]
        sems = [sem_a, sem_b]
        sizes = [GSB] * (NGS - 1) + [RPT - (NGS - 1) * GSB]

        def start(j):
            return pltpu.async_copy(
                x_hbm.at[spos_v.at[pl.ds(base + j * GSB, sizes[j])]],
                bufs[j % 2].at[pl.ds(0, sizes[j])], sems[j % 2],
            )

        cp = start(0)
        for j in range(NGS):
            cp.wait()
            if j + 1 < NGS:
                cp = start(j + 1)
            pltpu.sync_copy(
                bufs[j % 2].at[pl.ds(0, sizes[j])],
                rows_out.at[pl.ds(base + j * GSB, sizes[j])],
            )

    return k(x3, src, dst, val)


def _sc_combine(raw, g01, gx):
    info = plsc.get_sparse_core_info()
    nc = info.num_cores
    mesh = plsc.VectorSubcoreMesh(core_axis_name="c", subcore_axis_name="s")

    @functools.partial(
        pl.kernel,
        out_type=[
            jax.ShapeDtypeStruct((BS, D), jnp.float32),
            jax.ShapeDtypeStruct((BS, D), jnp.float32),
            jax.ShapeDtypeStruct((16, D), jnp.float32),
        ],
        mesh=mesh,
        scratch_types=[
            pltpu.VMEM((2, TPT), jnp.int32),
            pltpu.VMEM((2, 8), jnp.int32),
            pltpu.VMEM((CB, D), jnp.float32),
            pltpu.SemaphoreType.DMA,
        ],
    )
    def k(raw_hbm, g01_hbm, gx_hbm, out0_hbm, out1_hbm, ex_hbm,
          g_v, gx_v, buf_v, sem):
        wid = lax.axis_index("s") * nc + lax.axis_index("c")
        pltpu.sync_copy(g01_hbm.at[wid], g_v)
        for r in range(TPT // CB):
            base = wid * TPT + r * CB
            pltpu.async_copy(
                raw_hbm.at[g_v.at[0, pl.ds(r * CB, CB)]], buf_v, sem
            ).wait()
            pltpu.sync_copy(buf_v, out0_hbm.at[pl.ds(base, CB)])
            pltpu.async_copy(
                raw_hbm.at[g_v.at[1, pl.ds(r * CB, CB)]], buf_v, sem
            ).wait()
            pltpu.sync_copy(buf_v, out1_hbm.at[pl.ds(base, CB)])

        @pl.when(wid == 0)
        def _():
            pltpu.sync_copy(gx_hbm, gx_v)
            pltpu.async_copy(
                raw_hbm.at[gx_v.at[0]], buf_v.at[pl.ds(0, 8)], sem
            ).wait()
            pltpu.sync_copy(buf_v.at[pl.ds(0, 8)], ex_hbm.at[pl.ds(0, 8)])
            pltpu.async_copy(
                raw_hbm.at[gx_v.at[1]], buf_v.at[pl.ds(0, 8)], sem
            ).wait()
            pltpu.sync_copy(buf_v.at[pl.ds(0, 8)], ex_hbm.at[pl.ds(8, 8)])

    return k(raw, g01, gx)


def _fin_kernel(g0_ref, g1_ref, ex_ref, o_ref):
    blk = pl.program_id(0)
    o_ref[...] = g0_ref[...] + g1_ref[...]

    @pl.when(blk == 0)
    def _():
        o_ref[0:8, :] = o_ref[0:8, :] + (ex_ref[0:8, :] + ex_ref[8:16, :])


@jax.jit
def kernel(inputs, Wr, br, W1, b1, W2, b2):
    b, s, d = inputs.shape
    xf = inputs.reshape(BS, D)

    iw, w, a = pl.pallas_call(
        _router_kernel,
        grid=(NTB,),
        in_specs=[
            pl.BlockSpec((TB, D), lambda t: (t, 0)),
            pl.BlockSpec((E, D), lambda t: (0, 0)),
            pl.BlockSpec((1, E), lambda t: (0, 0)),
        ],
        out_specs=[
            pl.BlockSpec((TB, 128), lambda t: (t, 0)),
            pl.BlockSpec((TB, 128), lambda t: (t, 0)),
            pl.BlockSpec((8, 128), lambda t: (0, 0)),
        ],
        out_shape=[
            jax.ShapeDtypeStruct((BS, 128), jnp.int32),
            jax.ShapeDtypeStruct((BS, 128), jnp.float32),
            jax.ShapeDtypeStruct((8, 128), jnp.float32),
        ],
    )(xf, Wr, br.reshape(1, E))

    i1, i2 = iw[:, 0], iw[:, 1]
    pos1, pos2 = iw[:, 2], iw[:, 3]
    w1v, w2v = w[:, 0], w[:, 1]
    amass = a[0:2, 0:E]
    n_slots = (a[2, 0:E] + a[3, 0:E]).astype(jnp.int32)
    n_tot = n_slots + jnp.where(jnp.arange(E) < 2, 8, 0)
    padded = ((n_tot + BT - 1) // BT) * BT
    off = jnp.concatenate([jnp.zeros((1,), jnp.int32), jnp.cumsum(padded)[:-1]])
    cum_blk = jnp.cumsum(padded // BT)
    blk_expert = jnp.minimum(
        jnp.sum(
            (jnp.arange(NBLK, dtype=jnp.int32)[:, None] >= cum_blk[None, :]).astype(
                jnp.int32
            ),
            axis=1,
        ),
        E - 1,
    ).astype(jnp.int32)

    r8 = jnp.arange(8)
    i1_8, i2_8 = i1[:8], i2[:8]
    c0 = jnp.where(i1_8 < 2, amass[jnp.clip(i1_8, 0, 1), r8], 0.0)
    c1 = jnp.where(i2_8 < 2, amass[jnp.clip(i2_8, 0, 1), r8], 0.0)
    v0 = jnp.minimum(w1v.at[0:8].add(c0), CAPACITY)
    v1 = jnp.minimum(w2v.at[0:8].add(c1), CAPACITY)
    in_top = (i1_8[:, None] == jnp.arange(2)[None, :]) | (
        i2_8[:, None] == jnp.arange(2)[None, :]
    )
    vx = jnp.where(in_top, 0.0, jnp.minimum(amass.T, CAPACITY))
    dx = off[None, :2] + n_slots[None, :2] + r8[:, None]

    d0 = off[i1] + pos1
    d1 = off[i2] + pos2
    npad = PPAD - (2 * BS + 16)
    toks = jnp.arange(BS, dtype=jnp.int32)
    src = jnp.concatenate(
        [toks, toks, jnp.broadcast_to(r8[:, None], (8, 2)).reshape(-1),
         jnp.zeros((npad,), jnp.int32)]
    )
    dst = jnp.concatenate(
        [d0, d1, dx.reshape(-1), jnp.full((npad,), DUMMY, jnp.int32)]
    )
    val = jnp.concatenate(
        [v0, v1, vx.reshape(-1), jnp.zeros((npad,), jnp.float32)]
    )

    rows, vbuf = _sc_dispatch(xf, src, dst, val)

    w1b = W1.astype(jnp.bfloat16)
    w2b = W2.astype(jnp.bfloat16)
    raw = pl.pallas_call(
        _ffn_kernel,
        grid_spec=pltpu.PrefetchScalarGridSpec(
            num_scalar_prefetch=1,
            grid=(NBLK,),
            in_specs=[
                pl.BlockSpec((BT, D), lambda i, em: (i, 0)),
                pl.BlockSpec((BT, 1), lambda i, em: (i, 0)),
                pl.BlockSpec((1, FF, D), lambda i, em: (em[i], 0, 0)),
                pl.BlockSpec((1, 1, FF), lambda i, em: (em[i], 0, 0)),
                pl.BlockSpec((1, D, FF), lambda i, em: (em[i], 0, 0)),
                pl.BlockSpec((1, 1, D), lambda i, em: (em[i], 0, 0)),
            ],
            out_specs=pl.BlockSpec((BT, D), lambda i, em: (i, 0)),
        ),
        out_shape=jax.ShapeDtypeStruct((PAD, D), jnp.float32),
    )(
        blk_expert, rows, vbuf.reshape(PAD, 1), w1b, b1.reshape(E, 1, FF),
        w2b, b2.reshape(E, 1, D),
    )

    g01 = jnp.stack([d0, d1]).reshape(2, NW, TPT).transpose(1, 0, 2)
    gx = dx.T.astype(jnp.int32)

    g0rows, g1rows, exrows = _sc_combine(raw, g01, gx)
    FB = 512
    out = pl.pallas_call(
        _fin_kernel,
        grid=(BS // FB,),
        in_specs=[
            pl.BlockSpec((FB, D), lambda i: (i, 0)),
            pl.BlockSpec((FB, D), lambda i: (i, 0)),
            pl.BlockSpec((16, D), lambda i: (0, 0)),
        ],
        out_specs=pl.BlockSpec((FB, D), lambda i: (i, 0)),
        out_shape=jax.ShapeDtypeStruct((BS, D), jnp.float32),
    )(g0rows, g1rows, exrows)
    return out.reshape(b, s, d)
